# baseline (speedup 1.0000x reference)
import functools

import jax
import jax.numpy as jnp
from jax import lax
from jax.experimental import pallas as pl
from jax.experimental.pallas import tpu as pltpu
from jax.experimental.pallas import tpu_sc as plsc

_CONF = 0.25
_NMS = 0.45
_NCLS = 80
_IMG = 640.0
_N = 5000
_NP = 5120
_B = 2
_TJ = 512
_TI = 512
_NT = _NP // _TJ
_F = 16
_VF = 128
_SUP = 8192

_NW = 32
_ROWS = _B * _NP
_RPW = _ROWS // _NW
_CH = 64
_NCH = _RPW // _CH


def _prep_body(pred_ref, cp_ref, feat_ref, featr_ref):
    jt = pl.program_id(1)
    x = pred_ref[...]
    cp = cp_ref[...]
    rvalid = jt * _TJ + lax.broadcasted_iota(jnp.int32, (_TJ, 1), 0) < _N
    cxs = x[:, 0:1] * _IMG
    cys = x[:, 1:2] * _IMG
    ws = x[:, 2:3] * _IMG
    hs = x[:, 3:4] * _IMG
    x1 = cxs - ws / 2.0
    y1 = cys - hs / 2.0
    x2 = cxs + ws / 2.0
    y2 = cys + hs / 2.0
    li = lax.broadcasted_iota(jnp.int32, x.shape, 1)
    valid = (li >= 5) & (li < 5 + _NCLS)
    pm = jnp.where(valid, x, -jnp.inf)
    cls_conf = jnp.max(pm, axis=1, keepdims=True)
    idl = jnp.where(valid & (x == cls_conf), li - 5, 2**30)
    cls_id = jnp.min(idl, axis=1, keepdims=True)
    cls_f = cls_id.astype(jnp.float32)
    score = jnp.where(rvalid, x[:, 4:5] * cls_conf, 0.0)
    maskb = score > _CONF
    maskf = jnp.where(maskb, 1.0, 0.0)
    presf = jnp.max(jnp.where(cls_id == cp, 1.0, 0.0), axis=1, keepdims=True)
    off = cls_f * (2.0 * _IMG)
    ukey = lax.bitcast_convert_type(score, jnp.int32) * 2
    mwt = jnp.where(maskb, jnp.float32(_SUP + 1), jnp.float32(1.0))
    feat = jnp.concatenate(
        [x1 + off, y1 + off, x2 + off, y2 + off,
         score, cls_f, maskf, presf,
         x1, y1, x2, y2,
         lax.bitcast_convert_type(ukey, jnp.float32),
         mwt,
         jnp.zeros((_TJ, 2), jnp.float32)], axis=1)
    feat_ref[...] = feat
    featr_ref[...] = feat.T


def _pair_body(featc_ref, featr_ref, val_ref, idx_ref):
    b = pl.program_id(0)
    jt = pl.program_id(1)
    fc = featc_ref[...]
    x1j = fc[:, 0:1]
    y1j = fc[:, 1:2]
    x2j = fc[:, 2:3]
    y2j = fc[:, 3:4]
    ukj = lax.bitcast_convert_type(fc[:, 12:13], jnp.int32)
    areaj = (x2j - x1j) * (y2j - y1j)
    delta = (lax.broadcasted_iota(jnp.int32, (_TJ, _TI), 1)
             - lax.broadcasted_iota(jnp.int32, (_TJ, _TI), 0))
    ones = jnp.ones((_TI, 1), jnp.float32)
    acc = jnp.zeros((_TJ, 1), jnp.int32)
    for k in range(_NT):
        i0 = k * _TI
        fr = featr_ref[:, i0:i0 + _TI]
        x1i = fr[0:1, :]
        y1i = fr[1:2, :]
        x2i = fr[2:3, :]
        y2i = fr[3:4, :]
        uki = lax.bitcast_convert_type(fr[12:13, :], jnp.int32)
        mwi = fr[13:14, :]
        xx1 = jnp.maximum(x1i, x1j)
        yy1 = jnp.maximum(y1i, y1j)
        xx2 = jnp.minimum(x2i, x2j)
        yy2 = jnp.minimum(y2i, y2j)
        inter = jnp.clip(xx2 - xx1, 0.0) * jnp.clip(yy2 - yy1, 0.0)
        areai = (x2i - x1i) * (y2i - y1i)
        union = areai + areaj - inter
        iou = inter / jnp.maximum(union, 1e-9)
        ioug = iou > _NMS
        tie = jnp.where(delta < (jt - k) * _TJ, 1, 0)
        dom = (uki + tie) > ukj
        wi = jnp.where(ioug, mwi, 1.0)
        vj = jnp.where(dom, wi, 0.0)
        r = lax.dot_general(vj, ones, (((1,), (0,)), ((), ())),
                            precision=lax.Precision.HIGHEST,
                            preferred_element_type=jnp.float32)
        acc = acc + r.astype(jnp.int32)
    rank = jnp.bitwise_and(acc, _SUP - 1)
    keep = (fc[:, 6:7] > 0.5) & (acc < _SUP) & (fc[:, 7:8] > 0.5)
    finalf = jnp.where(keep, 1.0, 0.0)
    zpad = jnp.zeros((_TJ, _VF - 6), jnp.float32)
    val = jnp.concatenate(
        [fc[:, 8:12] * finalf, fc[:, 4:5] * finalf, fc[:, 5:6] * finalf,
         zpad], axis=1)
    val_ref[...] = val
    idx_ref[...] = b * _NP + rank


def _sc_scatter_body(val_hbm, idx_hbm, out_hbm, idx_v, rows_v, sem):
    wid = lax.axis_index("s") * 2 + lax.axis_index("c")
    base = wid * _RPW
    pltpu.sync_copy(idx_hbm.at[wid], idx_v)
    pltpu.sync_copy(val_hbm.at[pl.ds(base, _RPW)], rows_v)
    copies = [
        pltpu.async_copy(rows_v.at[pl.ds(c * _CH, _CH)],
                         out_hbm.at[idx_v.at[c]], sem)
        for c in range(_NCH)
    ]
    for c_ in copies:
        c_.wait()


def _tc_part(prediction, classes_present):
    cp = jnp.pad(classes_present.reshape(1, -1).astype(jnp.int32),
                 ((0, 0), (0, 128 - classes_present.shape[0])),
                 constant_values=-1)

    feat, featr = pl.pallas_call(
        _prep_body,
        grid=(_B, _NT),
        in_specs=[
            pl.BlockSpec((None, _TJ, prediction.shape[-1]),
                         lambda b, j: (b, j, 0)),
            pl.BlockSpec((1, 128), lambda b, j: (0, 0)),
        ],
        out_specs=[
            pl.BlockSpec((None, _TJ, _F), lambda b, j: (b, j, 0)),
            pl.BlockSpec((None, _F, _TJ), lambda b, j: (b, 0, j)),
        ],
        out_shape=[
            jax.ShapeDtypeStruct((_B, _NP, _F), jnp.float32),
            jax.ShapeDtypeStruct((_B, _F, _NP), jnp.float32),
        ],
    )(prediction.astype(jnp.float32), cp)

    val, idx = pl.pallas_call(
        _pair_body,
        grid=(_B, _NT),
        in_specs=[
            pl.BlockSpec((None, _TJ, _F), lambda b, j: (b, j, 0)),
            pl.BlockSpec((None, _F, _NP), lambda b, j: (b, 0, 0)),
        ],
        out_specs=[
            pl.BlockSpec((None, _TJ, _VF), lambda b, j: (b, j, 0)),
            pl.BlockSpec((None, _TJ, 1), lambda b, j: (b, j, 0)),
        ],
        out_shape=[
            jax.ShapeDtypeStruct((_B, _NP, _VF), jnp.float32),
            jax.ShapeDtypeStruct((_B, _NP, 1), jnp.int32),
        ],
    )(feat, featr)
    return val, idx


@jax.jit
def kernel(prediction, classes_present):
    val, idx = _tc_part(prediction, classes_present)
    val_flat = val.reshape(_ROWS, _VF)
    idx_flat = idx.reshape(_NW, _NCH, _CH)

    scatter = functools.partial(
        pl.kernel,
        mesh=plsc.VectorSubcoreMesh(core_axis_name="c", subcore_axis_name="s"),
        out_type=jax.ShapeDtypeStruct((_ROWS, _VF), jnp.float32),
        scratch_types=[
            pltpu.VMEM((_NCH, _CH), jnp.int32),
            pltpu.VMEM((_RPW, _VF), jnp.float32),
            pltpu.SemaphoreType.DMA,
        ],
    )(_sc_scatter_body)
    out = scatter(val_flat, idx_flat)

    return out.reshape(_B, _NP, _VF)[:, :_N, :6]


# speedup vs baseline: 1.3706x; 1.3706x over previous
import functools

import jax
import jax.numpy as jnp
from jax import lax
from jax.experimental import pallas as pl
from jax.experimental.pallas import tpu as pltpu
from jax.experimental.pallas import tpu_sc as plsc

_CONF = 0.25
_NMS = 0.45
_NCLS = 80
_IMG = 640.0
_N = 5000
_NP = 5120
_B = 2
_TJ = 512
_TI = 512
_NT = _NP // _TJ
_F = 16
_VF = 128
_SUP = 8192

_TP = 1024
_NW = 32
_ROWS = _B * _NP
_RPW = _ROWS // _NW
_CH = 64
_NCH = _RPW // _CH


def _prep_body(pred_ref, cp_ref, feat_ref, featr_ref):
    jt = pl.program_id(1)
    x = pred_ref[...]
    cp = cp_ref[...]
    rvalid = jt * _TJ + lax.broadcasted_iota(jnp.int32, (_TJ, 1), 0) < _N
    cxs = x[:, 0:1] * _IMG
    cys = x[:, 1:2] * _IMG
    ws = x[:, 2:3] * _IMG
    hs = x[:, 3:4] * _IMG
    x1 = cxs - ws / 2.0
    y1 = cys - hs / 2.0
    x2 = cxs + ws / 2.0
    y2 = cys + hs / 2.0
    li = lax.broadcasted_iota(jnp.int32, x.shape, 1)
    valid = (li >= 5) & (li < 5 + _NCLS)
    pm = jnp.where(valid, x, -jnp.inf)
    cls_conf = jnp.max(pm, axis=1, keepdims=True)
    idl = jnp.where(valid & (x == cls_conf), li - 5, 2**30)
    cls_id = jnp.min(idl, axis=1, keepdims=True)
    cls_f = cls_id.astype(jnp.float32)
    score = jnp.where(rvalid, x[:, 4:5] * cls_conf, 0.0)
    maskb = score > _CONF
    maskf = jnp.where(maskb, 1.0, 0.0)
    presf = jnp.max(jnp.where(cls_id == cp, 1.0, 0.0), axis=1, keepdims=True)
    off = cls_f * (2.0 * _IMG)
    ukey = lax.bitcast_convert_type(score, jnp.int32) * 2
    mwt = jnp.where(maskb, jnp.int32(_SUP + 1), jnp.int32(1))
    feat = jnp.concatenate(
        [x1 + off, y1 + off, x2 + off, y2 + off,
         score, cls_f, maskf, presf,
         x1, y1, x2, y2,
         lax.bitcast_convert_type(ukey, jnp.float32),
         lax.bitcast_convert_type(mwt, jnp.float32),
         jnp.zeros((_TJ, 2), jnp.float32)], axis=1)
    feat_ref[...] = feat
    featr_ref[...] = feat.T


def _pair_body(featc_ref, featr_ref, val_ref, idx_ref, accw_s):
    b = pl.program_id(0)
    jt = pl.program_id(1)
    fc = featc_ref[...]
    x1j = fc[:, 0:1]
    y1j = fc[:, 1:2]
    x2j = fc[:, 2:3]
    y2j = fc[:, 3:4]
    ukj = lax.bitcast_convert_type(fc[:, 12:13], jnp.int32)
    areaj = (x2j - x1j) * (y2j - y1j)
    delta = (lax.broadcasted_iota(jnp.int32, (_TP, _TI), 1)
             - lax.broadcasted_iota(jnp.int32, (_TP, _TI), 0))
    accw_s[...] = jnp.zeros((_TP, 128), jnp.int32)
    for k in range(_NT):
        i0 = k * _TI
        fr = featr_ref[:, i0:i0 + _TI]
        x1i = fr[0:1, :]
        y1i = fr[1:2, :]
        x2i = fr[2:3, :]
        y2i = fr[3:4, :]
        uki = lax.bitcast_convert_type(fr[12:13, :], jnp.int32)
        mwi = lax.bitcast_convert_type(fr[13:14, :], jnp.int32)
        xx1 = jnp.maximum(x1i, x1j)
        yy1 = jnp.maximum(y1i, y1j)
        xx2 = jnp.minimum(x2i, x2j)
        yy2 = jnp.minimum(y2i, y2j)
        inter = jnp.clip(xx2 - xx1, 0.0) * jnp.clip(yy2 - yy1, 0.0)
        areai = (x2i - x1i) * (y2i - y1i)
        union = areai + areaj - inter
        iou = inter / jnp.maximum(union, 1e-9)
        ioug = iou > _NMS
        tie = jnp.where(delta < jt * _TP - k * _TI, 1, 0)
        dom = (uki + tie) > ukj
        wi = jnp.where(ioug, mwi, 1)
        vj = jnp.where(dom, wi, 0)
        accw_s[...] += (vj[:, 0:128] + vj[:, 128:256]
                        + vj[:, 256:384] + vj[:, 384:512])
    acc = jnp.sum(accw_s[...], axis=1, keepdims=True)
    rank = jnp.bitwise_and(acc, _SUP - 1)
    keep = (fc[:, 6:7] > 0.5) & (acc < _SUP) & (fc[:, 7:8] > 0.5)
    finalf = jnp.where(keep, 1.0, 0.0)
    zpad = jnp.zeros((_TP, _VF - 6), jnp.float32)
    val = jnp.concatenate(
        [fc[:, 8:12] * finalf, fc[:, 4:5] * finalf, fc[:, 5:6] * finalf,
         zpad], axis=1)
    val_ref[...] = val
    idx_ref[...] = b * _NP + rank


def _sc_scatter_body(val_hbm, idx_hbm, out_hbm, idx_v, rows_v, sem):
    wid = lax.axis_index("s") * 2 + lax.axis_index("c")
    base = wid * _RPW
    pltpu.sync_copy(idx_hbm.at[wid], idx_v)
    pltpu.sync_copy(val_hbm.at[pl.ds(base, _RPW)], rows_v)
    copies = [
        pltpu.async_copy(rows_v.at[pl.ds(c * _CH, _CH)],
                         out_hbm.at[idx_v.at[c]], sem)
        for c in range(_NCH)
    ]
    for c_ in copies:
        c_.wait()


def _tc_part(prediction, classes_present):
    cp = jnp.pad(classes_present.reshape(1, -1).astype(jnp.int32),
                 ((0, 0), (0, 128 - classes_present.shape[0])),
                 constant_values=-1)

    feat, featr = pl.pallas_call(
        _prep_body,
        grid=(_B, _NT),
        in_specs=[
            pl.BlockSpec((None, _TJ, prediction.shape[-1]),
                         lambda b, j: (b, j, 0)),
            pl.BlockSpec((1, 128), lambda b, j: (0, 0)),
        ],
        out_specs=[
            pl.BlockSpec((None, _TJ, _F), lambda b, j: (b, j, 0)),
            pl.BlockSpec((None, _F, _TJ), lambda b, j: (b, 0, j)),
        ],
        out_shape=[
            jax.ShapeDtypeStruct((_B, _NP, _F), jnp.float32),
            jax.ShapeDtypeStruct((_B, _F, _NP), jnp.float32),
        ],
    )(prediction.astype(jnp.float32), cp)

    val, idx = pl.pallas_call(
        _pair_body,
        grid=(_B, _NP // _TP),
        in_specs=[
            pl.BlockSpec((None, _TP, _F), lambda b, j: (b, j, 0)),
            pl.BlockSpec((None, _F, _NP), lambda b, j: (b, 0, 0)),
        ],
        out_specs=[
            pl.BlockSpec((None, _TP, _VF), lambda b, j: (b, j, 0)),
            pl.BlockSpec((None, _TP, 1), lambda b, j: (b, j, 0)),
        ],
        out_shape=[
            jax.ShapeDtypeStruct((_B, _NP, _VF), jnp.float32),
            jax.ShapeDtypeStruct((_B, _NP, 1), jnp.int32),
        ],
        scratch_shapes=[pltpu.VMEM((_TP, 128), jnp.int32)],
    )(feat, featr)
    return val, idx


@jax.jit
def kernel(prediction, classes_present):
    val, idx = _tc_part(prediction, classes_present)
    val_flat = val.reshape(_ROWS, _VF)
    idx_flat = idx.reshape(_NW, _NCH, _CH)

    scatter = functools.partial(
        pl.kernel,
        mesh=plsc.VectorSubcoreMesh(core_axis_name="c", subcore_axis_name="s"),
        out_type=jax.ShapeDtypeStruct((_ROWS, _VF), jnp.float32),
        scratch_types=[
            pltpu.VMEM((_NCH, _CH), jnp.int32),
            pltpu.VMEM((_RPW, _VF), jnp.float32),
            pltpu.SemaphoreType.DMA,
        ],
    )(_sc_scatter_body)
    out = scatter(val_flat, idx_flat)

    return out.reshape(_B, _NP, _VF)[:, :_N, :6]
